# baseline (speedup 1.0000x reference)
import functools

import jax
import jax.numpy as jnp
from jax import lax
from jax.experimental import pallas as pl
from jax.experimental.pallas import tpu as pltpu
from jax.experimental.pallas import tpu_sc as plsc

_ROWS = 4096
_COLS = 8192
_XC = 1024
_TOPK = 64
_INT_MIN = -(2 ** 31)
_MASK = 0x7FFFFFFF

_NW = 32
_SC_ROWS = 2304
_RPW = _SC_ROWS // _NW
_NVEC = _COLS // 16
_U = 8

_TC_ROWS = _ROWS - _SC_ROWS
_BLOCK_R = 256
_GRID = _TC_ROWS // _BLOCK_R


def _tokey(v):
    b = lax.bitcast_convert_type(v, jnp.int32)
    return jnp.where(b >= 0, b, b ^ _MASK)


def _toval(k):
    b = jnp.where(k >= 0, k, k ^ _MASK)
    return lax.bitcast_convert_type(b, jnp.float32)


def _sc_body(attn_hbm, out_hbm, rowbuf, keybuf, outvec, sem):
    c = lax.axis_index("c")
    s = lax.axis_index("s")
    wid = s * 2 + c
    base_row = wid * _RPW

    pltpu.async_copy(attn_hbm.at[base_row], rowbuf.at[pl.ds(0, _COLS)], sem)

    def row_body(r, carry):
        s2_acc, top_acc, bprev = carry
        par = (r & 1) * _COLS
        pltpu.make_async_copy(
            attn_hbm.at[base_row], rowbuf.at[pl.ds(par, _COLS)], sem).wait()

        @pl.when(r + 1 < _RPW)
        def _():
            npar = ((r + 1) & 1) * _COLS
            pltpu.async_copy(attn_hbm.at[base_row + r + 1],
                             rowbuf.at[pl.ds(npar, _COLS)], sem)

        gms = []
        ssqs = [jnp.zeros((16,), jnp.float32) for _ in range(2)]
        off = jnp.zeros((16,), jnp.int32)
        for g in range(4):
            def p1(j, cr, g=g):
                s0, s1, m0, m1, base = cr
                vs, ms, css, pcs = [], [], [], []
                for u in range(_U):
                    v = rowbuf[pl.ds(par + (g * 128 + j * _U + u) * 16, 16)]
                    vs.append(v)
                    m = v >= bprev
                    ms.append(m)
                    css.append(plsc.cumsum(m.astype(jnp.int32)))
                    pcs.append(plsc.all_reduce_population_count(m))
                    if u % 2 == 0:
                        s0 = s0 + v * v
                        m0 = jnp.maximum(m0, v)
                    else:
                        s1 = s1 + v * v
                        m1 = jnp.maximum(m1, v)
                for u in range(_U):
                    plsc.store_scatter(keybuf, [base + css[u] - 1],
                                       _tokey(vs[u]), mask=ms[u])
                    base = base + pcs[u]
                return s0, s1, m0, m1, base
            gm0 = jnp.full((16,), -jnp.inf, jnp.float32)
            s0, s1, m0, m1, off = lax.fori_loop(
                0, 128 // _U, p1, (ssqs[0], ssqs[1], gm0, gm0, off))
            ssqs = [s0, s1]
            gms.append(jnp.maximum(m0, m1))
        gmin = jnp.minimum(jnp.minimum(gms[0], gms[1]),
                           jnp.minimum(gms[2], gms[3]))
        gmax = jnp.maximum(jnp.maximum(gms[0], gms[1]),
                           jnp.maximum(gms[2], gms[3]))
        lval = -jnp.max(-gmin)
        mval = jnp.max(gmax)
        s2_row = jnp.sum(ssqs[0] + ssqs[1])
        nc_spec = jnp.max(off)

        def p2fb(j, off2):
            vs, ms, css, pcs = [], [], [], []
            for u in range(_U):
                v = rowbuf[pl.ds(par + (j * _U + u) * 16, 16)]
                m = v >= lval
                vs.append(v)
                ms.append(m)
                css.append(plsc.cumsum(m.astype(jnp.int32)))
                pcs.append(plsc.all_reduce_population_count(m))
            base = off2
            for u in range(_U):
                plsc.store_scatter(keybuf, [base + css[u] - 1],
                                   _tokey(vs[u]), mask=ms[u])
                base = base + pcs[u]
            return base

        def fb(_):
            off2 = lax.fori_loop(0, _NVEC // _U, p2fb,
                                 jnp.zeros((16,), jnp.int32))
            return jnp.max(off2), lval

        nc, lo_val = lax.cond(nc_spec < _TOPK, fb,
                              lambda _: (nc_spec, bprev), None)

        sent = jnp.full((16,), jnp.int32(_INT_MIN))
        lane16 = lax.broadcasted_iota(jnp.int32, (16,), 0)
        tmask = jnp.ones((16,), jnp.bool_)
        for u in range(4):
            plsc.store_scatter(keybuf, [nc + u * 16 + lane16], sent,
                               mask=tmask)
        nv4 = (nc + 63) >> 6

        klo = _tokey(lo_val)
        khi = _tokey(mval)
        u_lo = klo ^ jnp.int32(_INT_MIN)
        u_hi = khi ^ jnp.int32(_INT_MIN)
        sm = u_lo ^ u_hi
        sm = sm | lax.shift_right_logical(sm, 1)
        sm = sm | lax.shift_right_logical(sm, 2)
        sm = sm | lax.shift_right_logical(sm, 4)
        sm = sm | lax.shift_right_logical(sm, 8)
        sm = sm | lax.shift_right_logical(sm, 16)
        sp1 = sm + 1
        expo = lax.shift_right_logical(
            lax.bitcast_convert_type(sp1.astype(jnp.float32), jnp.int32),
            23) - 127
        nbits = jnp.where(sm < 0, jnp.int32(32),
                          jnp.where(sm == 0, jnp.int32(0), expo))

        def sbody(i, u):
            bit = jnp.int32(1) << (nbits - 1 - i)
            cand = u | bit
            thr = cand ^ jnp.int32(_INT_MIN)

            def cbody(jj, cr):
                a0, a1 = cr
                k0 = keybuf[pl.ds((jj * 4) * 16, 16)]
                k1 = keybuf[pl.ds((jj * 4 + 1) * 16, 16)]
                k2 = keybuf[pl.ds((jj * 4 + 2) * 16, 16)]
                k3 = keybuf[pl.ds((jj * 4 + 3) * 16, 16)]
                a0 = a0 + (k0 >= thr).astype(jnp.int32) \
                    + (k2 >= thr).astype(jnp.int32)
                a1 = a1 + (k1 >= thr).astype(jnp.int32) \
                    + (k3 >= thr).astype(jnp.int32)
                return a0, a1
            z = jnp.zeros((16,), jnp.int32)
            a0, a1 = lax.fori_loop(0, nv4, cbody, (z, z))
            cnt = jnp.sum(a0 + a1)
            return jnp.where(cnt >= _TOPK, cand, u)

        u_fin = lax.fori_loop(0, nbits, sbody, u_hi & ~sm)
        t_key = u_fin ^ jnp.int32(_INT_MIN)

        def fbody(jj, cr):
            cgt, sgt = cr
            for u4 in range(4):
                kv = keybuf[pl.ds((jj * 4 + u4) * 16, 16)]
                gt = kv > t_key
                fv = _toval(kv)
                sgt = sgt + jnp.where(gt, fv * fv, 0.0)
                cgt = cgt + gt.astype(jnp.int32)
            return cgt, sgt
        cgt, sgt = lax.fori_loop(
            0, nv4, fbody,
            (jnp.zeros((16,), jnp.int32), jnp.zeros((16,), jnp.float32)))
        cnt_gt = jnp.sum(cgt)
        ssq_gt = jnp.sum(sgt)
        tval = _toval(t_key)
        top_row = ssq_gt + (_TOPK - cnt_gt).astype(jnp.float32) * tval * tval
        return s2_acc + s2_row, top_acc + top_row, tval

    s2_acc, top_acc, _ = lax.fori_loop(
        0, _RPW, row_body,
        (jnp.float32(0.0), jnp.float32(0.0), jnp.float32(jnp.inf)))

    lane = lax.broadcasted_iota(jnp.int32, (16,), 0)
    ov = jnp.where(lane == 0, s2_acc, jnp.where(lane == 1, top_acc, 0.0))
    outvec[...] = ov
    pltpu.sync_copy(outvec, out_hbm.at[wid])


_sc_topk = functools.partial(
    pl.kernel,
    mesh=plsc.VectorSubcoreMesh(core_axis_name="c", subcore_axis_name="s"),
    out_type=jax.ShapeDtypeStruct((_NW, 16), jnp.float32),
    scratch_types=[
        pltpu.VMEM((2 * _COLS,), jnp.float32),
        pltpu.VMEM((_COLS + 64,), jnp.int32),
        pltpu.VMEM((16,), jnp.float32),
        pltpu.SemaphoreType.DMA,
    ],
    compiler_params=pltpu.CompilerParams(needs_layout_passes=False),
)(_sc_body)


def _count16(mask16):
    m = mask16
    while m.shape[1] > 128:
        h = m.shape[1] // 2
        m = m[:, :h] + m[:, h:]
    return jnp.sum(m.astype(jnp.int32), axis=1, keepdims=True)


def _tc_block(a_ref, s2_ref, top_ref):
    a = a_ref[...]
    sq = a * a
    s2_ref[...] = jnp.sum(sq).reshape(1, 1, 1)

    b = jax.lax.bitcast_convert_type(a, jnp.int32)
    key = jnp.where(b >= 0, b, b ^ _MASK)

    key3 = key.reshape(a.shape[0], _COLS // 128, 128)
    cmax = jnp.max(key3, axis=1)
    lo_key = jnp.min(cmax)
    hi_key = jnp.max(cmax)
    u_lo = lo_key ^ jnp.int32(_INT_MIN)
    u_hi = hi_key ^ jnp.int32(_INT_MIN)
    sm = u_lo ^ u_hi
    sm = sm | jax.lax.shift_right_logical(sm, 1)
    sm = sm | jax.lax.shift_right_logical(sm, 2)
    sm = sm | jax.lax.shift_right_logical(sm, 4)
    sm = sm | jax.lax.shift_right_logical(sm, 8)
    sm = sm | jax.lax.shift_right_logical(sm, 16)
    sp1 = sm + 1
    expo = jax.lax.shift_right_logical(
        jax.lax.bitcast_convert_type(sp1.astype(jnp.float32), jnp.int32),
        23) - 127
    nbits = jnp.where(sm < 0, jnp.int32(32),
                      jnp.where(sm == 0, jnp.int32(0), expo))
    u_start = u_hi & ~sm

    k16 = (key >> 16).astype(jnp.int16)
    nh = jnp.maximum(nbits - 16, 0)
    uh_start = jax.lax.shift_right_logical(u_start, 16)

    def body_hi(i, uh):
        bit = jnp.int32(1) << (nh - 1 - i)
        cand = uh | bit
        t16c = (cand ^ 0x8000).astype(jnp.int16)
        cnt = _count16((k16 >= t16c).astype(jnp.int16))
        return jnp.where(cnt >= _TOPK, cand, uh)

    uh0 = jnp.full((a.shape[0], 1), uh_start, jnp.int32)
    uh = jax.lax.fori_loop(0, nh, body_hi, uh0)
    t16 = (uh ^ 0x8000).astype(jnp.int16)

    c_gt_hi = _count16((k16 > t16).astype(jnp.int16))
    r = _TOPK - c_gt_hi

    l16s = key.astype(jnp.int16) ^ jnp.int16(-(2 ** 15))
    m16 = jnp.where(k16 == t16, l16s, jnp.int16(-(2 ** 15)))
    nl = jnp.minimum(nbits, 16)
    ul_start = u_start & 0xFFFF & ~((jnp.int32(1) << nl) - 1)

    def body_lo(i, ul):
        bit = jnp.int32(1) << (nl - 1 - i)
        cand = ul | bit
        ts = (cand ^ 0x8000).astype(jnp.int16)
        cnt = _count16((m16 >= ts).astype(jnp.int16))
        return jnp.where(cnt >= r, cand, ul)

    ul0 = jnp.full((a.shape[0], 1), ul_start, jnp.int32)
    ul = jax.lax.fori_loop(0, nl, body_lo, ul0)

    t_key = (jax.lax.convert_element_type(t16, jnp.int32) << 16) | ul

    gt = key > t_key
    cnt_gt = jnp.sum(gt.astype(jnp.float32), axis=1, keepdims=True)
    ssq_gt = jnp.sum(jnp.where(gt, sq, 0.0), axis=1, keepdims=True)
    t_val = _toval(t_key)
    top = ssq_gt + (_TOPK - cnt_gt) * t_val * t_val
    top_ref[...] = jnp.sum(top).reshape(1, 1, 1)


def _rec_block(x_ref, y_ref, rec_ref):
    d = x_ref[...] - y_ref[...]
    rec_ref[...] = jnp.sum(d * d).reshape(1, 1, 1)


def kernel(x, y, attn):
    sc_parts = _sc_topk(attn)
    s2_p, top_p = pl.pallas_call(
        _tc_block,
        grid=(_GRID,),
        in_specs=[pl.BlockSpec(
            (_BLOCK_R, _COLS),
            lambda i: (i + _SC_ROWS // _BLOCK_R, 0))],
        out_specs=[
            pl.BlockSpec((1, 1, 1), lambda i: (i, 0, 0)),
            pl.BlockSpec((1, 1, 1), lambda i: (i, 0, 0)),
        ],
        out_shape=[
            jax.ShapeDtypeStruct((_GRID, 1, 1), jnp.float32),
            jax.ShapeDtypeStruct((_GRID, 1, 1), jnp.float32),
        ],
    )(attn)
    rec_p = pl.pallas_call(
        _rec_block,
        grid=(4,),
        in_specs=[
            pl.BlockSpec((_ROWS // 4, _XC), lambda i: (i, 0)),
            pl.BlockSpec((_ROWS // 4, _XC), lambda i: (i, 0)),
        ],
        out_specs=pl.BlockSpec((1, 1, 1), lambda i: (i, 0, 0)),
        out_shape=jax.ShapeDtypeStruct((4, 1, 1), jnp.float32),
    )(x, y)
    rec = jnp.sum(rec_p) / (_ROWS * _XC)
    s2 = jnp.sum(s2_p) + jnp.sum(sc_parts[:, 0])
    top = jnp.sum(top_p) + jnp.sum(sc_parts[:, 1])
    attn_loss = (s2 - top) / (_ROWS * _COLS)
    return rec + 0.5 * attn_loss


# speedup vs baseline: 1.1841x; 1.1841x over previous
import functools

import jax
import jax.numpy as jnp
from jax import lax
from jax.experimental import pallas as pl
from jax.experimental.pallas import tpu as pltpu
from jax.experimental.pallas import tpu_sc as plsc

_ROWS = 4096
_COLS = 8192
_XC = 1024
_TOPK = 64
_INT_MIN = -(2 ** 31)
_MASK = 0x7FFFFFFF

_NW = 32
_SC_ROWS = 2304
_RPW = _SC_ROWS // _NW
_NVEC = _COLS // 16
_U = 8

_TC_ROWS = _ROWS - _SC_ROWS
_BLOCK_R = 256
_GRID = _TC_ROWS // _BLOCK_R


def _tokey(v):
    b = lax.bitcast_convert_type(v, jnp.int32)
    return jnp.where(b >= 0, b, b ^ _MASK)


def _toval(k):
    b = jnp.where(k >= 0, k, k ^ _MASK)
    return lax.bitcast_convert_type(b, jnp.float32)


def _sc_body(attn_hbm, out_hbm, rowbuf, keybuf, outvec, sem):
    c = lax.axis_index("c")
    s = lax.axis_index("s")
    wid = s * 2 + c
    base_row = wid * _RPW

    pltpu.async_copy(attn_hbm.at[base_row], rowbuf.at[pl.ds(0, _COLS)], sem)

    def row_body(r, carry):
        s2_acc, top_acc, bprev = carry
        par = (r & 1) * _COLS
        pltpu.make_async_copy(
            attn_hbm.at[base_row], rowbuf.at[pl.ds(par, _COLS)], sem).wait()

        @pl.when(r + 1 < _RPW)
        def _():
            npar = ((r + 1) & 1) * _COLS
            pltpu.async_copy(attn_hbm.at[base_row + r + 1],
                             rowbuf.at[pl.ds(npar, _COLS)], sem)

        gms = []
        ssqs = [jnp.zeros((16,), jnp.float32) for _ in range(2)]
        off = jnp.zeros((16,), jnp.int32)
        for g in range(4):
            def p1(j, cr, g=g):
                s0, s1, m0, m1, base = cr
                vs, ms, css, pcs = [], [], [], []
                for u in range(_U):
                    v = rowbuf[pl.ds(par + (g * 128 + j * _U + u) * 16, 16)]
                    vs.append(v)
                    m = v >= bprev
                    ms.append(m)
                    css.append(plsc.cumsum(m.astype(jnp.int32)))
                    pcs.append(plsc.all_reduce_population_count(m))
                    if u % 2 == 0:
                        s0 = s0 + v * v
                        m0 = jnp.maximum(m0, v)
                    else:
                        s1 = s1 + v * v
                        m1 = jnp.maximum(m1, v)
                for u in range(_U):
                    plsc.store_scatter(keybuf, [base + css[u] - 1],
                                       _tokey(vs[u]), mask=ms[u])
                    base = base + pcs[u]
                return s0, s1, m0, m1, base
            gm0 = jnp.full((16,), -jnp.inf, jnp.float32)
            s0, s1, m0, m1, off = lax.fori_loop(
                0, 128 // _U, p1, (ssqs[0], ssqs[1], gm0, gm0, off))
            ssqs = [s0, s1]
            gms.append(jnp.maximum(m0, m1))
        gmin = jnp.minimum(jnp.minimum(gms[0], gms[1]),
                           jnp.minimum(gms[2], gms[3]))
        gmax = jnp.maximum(jnp.maximum(gms[0], gms[1]),
                           jnp.maximum(gms[2], gms[3]))
        lval = -jnp.max(-gmin)
        mval = jnp.max(gmax)
        s2_row = jnp.sum(ssqs[0] + ssqs[1])
        nc_spec = jnp.max(off)

        def p2fb(j, off2):
            vs, ms, css, pcs = [], [], [], []
            for u in range(_U):
                v = rowbuf[pl.ds(par + (j * _U + u) * 16, 16)]
                m = v >= lval
                vs.append(v)
                ms.append(m)
                css.append(plsc.cumsum(m.astype(jnp.int32)))
                pcs.append(plsc.all_reduce_population_count(m))
            base = off2
            for u in range(_U):
                plsc.store_scatter(keybuf, [base + css[u] - 1],
                                   _tokey(vs[u]), mask=ms[u])
                base = base + pcs[u]
            return base

        def fb(_):
            off2 = lax.fori_loop(0, _NVEC // _U, p2fb,
                                 jnp.zeros((16,), jnp.int32))
            return jnp.max(off2), lval

        nc, lo_val = lax.cond(nc_spec < _TOPK, fb,
                              lambda _: (nc_spec, bprev), None)

        sent = jnp.full((16,), jnp.int32(_INT_MIN))
        lane16 = lax.broadcasted_iota(jnp.int32, (16,), 0)
        tmask = jnp.ones((16,), jnp.bool_)
        for u in range(4):
            plsc.store_scatter(keybuf, [nc + u * 16 + lane16], sent,
                               mask=tmask)
        nv4 = (nc + 63) >> 6

        klo = _tokey(lo_val)
        khi = _tokey(mval)
        u_lo = klo ^ jnp.int32(_INT_MIN)
        u_hi = khi ^ jnp.int32(_INT_MIN)
        sm = u_lo ^ u_hi
        sm = sm | lax.shift_right_logical(sm, 1)
        sm = sm | lax.shift_right_logical(sm, 2)
        sm = sm | lax.shift_right_logical(sm, 4)
        sm = sm | lax.shift_right_logical(sm, 8)
        sm = sm | lax.shift_right_logical(sm, 16)
        sp1 = sm + 1
        expo = lax.shift_right_logical(
            lax.bitcast_convert_type(sp1.astype(jnp.float32), jnp.int32),
            23) - 127
        nbits = jnp.where(sm < 0, jnp.int32(32),
                          jnp.where(sm == 0, jnp.int32(0), expo))

        def sbody(i, u):
            bit = jnp.int32(1) << (nbits - 1 - i)
            cand = u | bit
            thr = cand ^ jnp.int32(_INT_MIN)

            def cbody(jj, cr):
                a0, a1 = cr
                k0 = keybuf[pl.ds((jj * 4) * 16, 16)]
                k1 = keybuf[pl.ds((jj * 4 + 1) * 16, 16)]
                k2 = keybuf[pl.ds((jj * 4 + 2) * 16, 16)]
                k3 = keybuf[pl.ds((jj * 4 + 3) * 16, 16)]
                a0 = a0 + (k0 >= thr).astype(jnp.int32) \
                    + (k2 >= thr).astype(jnp.int32)
                a1 = a1 + (k1 >= thr).astype(jnp.int32) \
                    + (k3 >= thr).astype(jnp.int32)
                return a0, a1
            z = jnp.zeros((16,), jnp.int32)
            a0, a1 = lax.fori_loop(0, nv4, cbody, (z, z))
            cnt = jnp.sum(a0 + a1)
            return jnp.where(cnt >= _TOPK, cand, u)

        u_fin = lax.fori_loop(0, nbits, sbody, u_hi & ~sm)
        t_key = u_fin ^ jnp.int32(_INT_MIN)

        def fbody(jj, cr):
            cgt, sgt = cr
            for u4 in range(4):
                kv = keybuf[pl.ds((jj * 4 + u4) * 16, 16)]
                gt = kv > t_key
                fv = _toval(kv)
                sgt = sgt + jnp.where(gt, fv * fv, 0.0)
                cgt = cgt + gt.astype(jnp.int32)
            return cgt, sgt
        cgt, sgt = lax.fori_loop(
            0, nv4, fbody,
            (jnp.zeros((16,), jnp.int32), jnp.zeros((16,), jnp.float32)))
        cnt_gt = jnp.sum(cgt)
        ssq_gt = jnp.sum(sgt)
        tval = _toval(t_key)
        top_row = ssq_gt + (_TOPK - cnt_gt).astype(jnp.float32) * tval * tval
        return s2_acc + s2_row, top_acc + top_row, lval

    s2_acc, top_acc, _ = lax.fori_loop(
        0, _RPW, row_body,
        (jnp.float32(0.0), jnp.float32(0.0), jnp.float32(jnp.inf)))

    lane = lax.broadcasted_iota(jnp.int32, (16,), 0)
    ov = jnp.where(lane == 0, s2_acc, jnp.where(lane == 1, top_acc, 0.0))
    outvec[...] = ov
    pltpu.sync_copy(outvec, out_hbm.at[wid])


_sc_topk = functools.partial(
    pl.kernel,
    mesh=plsc.VectorSubcoreMesh(core_axis_name="c", subcore_axis_name="s"),
    out_type=jax.ShapeDtypeStruct((_NW, 16), jnp.float32),
    scratch_types=[
        pltpu.VMEM((2 * _COLS,), jnp.float32),
        pltpu.VMEM((_COLS + 64,), jnp.int32),
        pltpu.VMEM((16,), jnp.float32),
        pltpu.SemaphoreType.DMA,
    ],
    compiler_params=pltpu.CompilerParams(needs_layout_passes=False),
)(_sc_body)


def _count16(mask16):
    m = mask16
    while m.shape[1] > 128:
        h = m.shape[1] // 2
        m = m[:, :h] + m[:, h:]
    return jnp.sum(m.astype(jnp.int32), axis=1, keepdims=True)


def _tc_block(a_ref, s2_ref, top_ref):
    a = a_ref[...]
    sq = a * a
    s2_ref[...] = jnp.sum(sq).reshape(1, 1, 1)

    b = jax.lax.bitcast_convert_type(a, jnp.int32)
    key = jnp.where(b >= 0, b, b ^ _MASK)

    key3 = key.reshape(a.shape[0], _COLS // 128, 128)
    cmax = jnp.max(key3, axis=1)
    lo_key = jnp.min(cmax)
    hi_key = jnp.max(cmax)
    u_lo = lo_key ^ jnp.int32(_INT_MIN)
    u_hi = hi_key ^ jnp.int32(_INT_MIN)
    sm = u_lo ^ u_hi
    sm = sm | jax.lax.shift_right_logical(sm, 1)
    sm = sm | jax.lax.shift_right_logical(sm, 2)
    sm = sm | jax.lax.shift_right_logical(sm, 4)
    sm = sm | jax.lax.shift_right_logical(sm, 8)
    sm = sm | jax.lax.shift_right_logical(sm, 16)
    sp1 = sm + 1
    expo = jax.lax.shift_right_logical(
        jax.lax.bitcast_convert_type(sp1.astype(jnp.float32), jnp.int32),
        23) - 127
    nbits = jnp.where(sm < 0, jnp.int32(32),
                      jnp.where(sm == 0, jnp.int32(0), expo))
    u_start = u_hi & ~sm

    k16 = (key >> 16).astype(jnp.int16)
    nh = jnp.maximum(nbits - 16, 0)
    uh_start = jax.lax.shift_right_logical(u_start, 16)

    def body_hi(i, uh):
        bit = jnp.int32(1) << (nh - 1 - i)
        cand = uh | bit
        t16c = (cand ^ 0x8000).astype(jnp.int16)
        cnt = _count16((k16 >= t16c).astype(jnp.int16))
        return jnp.where(cnt >= _TOPK, cand, uh)

    uh0 = jnp.full((a.shape[0], 1), uh_start, jnp.int32)
    uh = jax.lax.fori_loop(0, nh, body_hi, uh0)
    t16 = (uh ^ 0x8000).astype(jnp.int16)

    c_gt_hi = _count16((k16 > t16).astype(jnp.int16))
    r = _TOPK - c_gt_hi

    l16s = key.astype(jnp.int16) ^ jnp.int16(-(2 ** 15))
    m16 = jnp.where(k16 == t16, l16s, jnp.int16(-(2 ** 15)))
    nl = jnp.minimum(nbits, 16)
    ul_start = u_start & 0xFFFF & ~((jnp.int32(1) << nl) - 1)

    def body_lo(i, ul):
        bit = jnp.int32(1) << (nl - 1 - i)
        cand = ul | bit
        ts = (cand ^ 0x8000).astype(jnp.int16)
        cnt = _count16((m16 >= ts).astype(jnp.int16))
        return jnp.where(cnt >= r, cand, ul)

    ul0 = jnp.full((a.shape[0], 1), ul_start, jnp.int32)
    ul = jax.lax.fori_loop(0, nl, body_lo, ul0)

    t_key = (jax.lax.convert_element_type(t16, jnp.int32) << 16) | ul

    gt = key > t_key
    cnt_gt = jnp.sum(gt.astype(jnp.float32), axis=1, keepdims=True)
    ssq_gt = jnp.sum(jnp.where(gt, sq, 0.0), axis=1, keepdims=True)
    t_val = _toval(t_key)
    top = ssq_gt + (_TOPK - cnt_gt) * t_val * t_val
    top_ref[...] = jnp.sum(top).reshape(1, 1, 1)


def _rec_block(x_ref, y_ref, rec_ref):
    d = x_ref[...] - y_ref[...]
    rec_ref[...] = jnp.sum(d * d).reshape(1, 1, 1)


def kernel(x, y, attn):
    sc_parts = _sc_topk(attn)
    s2_p, top_p = pl.pallas_call(
        _tc_block,
        grid=(_GRID,),
        in_specs=[pl.BlockSpec(
            (_BLOCK_R, _COLS),
            lambda i: (i + _SC_ROWS // _BLOCK_R, 0))],
        out_specs=[
            pl.BlockSpec((1, 1, 1), lambda i: (i, 0, 0)),
            pl.BlockSpec((1, 1, 1), lambda i: (i, 0, 0)),
        ],
        out_shape=[
            jax.ShapeDtypeStruct((_GRID, 1, 1), jnp.float32),
            jax.ShapeDtypeStruct((_GRID, 1, 1), jnp.float32),
        ],
    )(attn)
    rec_p = pl.pallas_call(
        _rec_block,
        grid=(4,),
        in_specs=[
            pl.BlockSpec((_ROWS // 4, _XC), lambda i: (i, 0)),
            pl.BlockSpec((_ROWS // 4, _XC), lambda i: (i, 0)),
        ],
        out_specs=pl.BlockSpec((1, 1, 1), lambda i: (i, 0, 0)),
        out_shape=jax.ShapeDtypeStruct((4, 1, 1), jnp.float32),
    )(x, y)
    rec = jnp.sum(rec_p) / (_ROWS * _XC)
    s2 = jnp.sum(s2_p) + jnp.sum(sc_parts[:, 0])
    top = jnp.sum(top_p) + jnp.sum(sc_parts[:, 1])
    attn_loss = (s2 - top) / (_ROWS * _COLS)
    return rec + 0.5 * attn_loss


# speedup vs baseline: 1.2250x; 1.0345x over previous
import functools

import jax
import jax.numpy as jnp
from jax import lax
from jax.experimental import pallas as pl
from jax.experimental.pallas import tpu as pltpu
from jax.experimental.pallas import tpu_sc as plsc

_ROWS = 4096
_COLS = 8192
_XC = 1024
_TOPK = 64
_INT_MIN = -(2 ** 31)
_MASK = 0x7FFFFFFF

_NW = 32
_SC_ROWS = 2304
_RPW = _SC_ROWS // _NW
_NVEC = _COLS // 16
_U = 8

_TC_ROWS = _ROWS - _SC_ROWS
_BLOCK_R = 256
_GRID = _TC_ROWS // _BLOCK_R


def _tokey(v):
    b = lax.bitcast_convert_type(v, jnp.int32)
    return jnp.where(b >= 0, b, b ^ _MASK)


def _toval(k):
    b = jnp.where(k >= 0, k, k ^ _MASK)
    return lax.bitcast_convert_type(b, jnp.float32)


def _sc_body(attn_hbm, out_hbm, rowbuf, keybuf, outvec, sem):
    c = lax.axis_index("c")
    s = lax.axis_index("s")
    wid = s * 2 + c
    base_row = wid * _RPW

    pltpu.async_copy(attn_hbm.at[base_row], rowbuf.at[pl.ds(0, _COLS)], sem)

    def row_body(r, carry):
        s2_acc, top_acc, bprev = carry
        par = (r & 1) * _COLS
        pltpu.make_async_copy(
            attn_hbm.at[base_row], rowbuf.at[pl.ds(par, _COLS)], sem).wait()

        @pl.when(r + 1 < _RPW)
        def _():
            npar = ((r + 1) & 1) * _COLS
            pltpu.async_copy(attn_hbm.at[base_row + r + 1],
                             rowbuf.at[pl.ds(npar, _COLS)], sem)

        gms = []
        ssqs = [jnp.zeros((16,), jnp.float32) for _ in range(2)]
        off = jnp.zeros((16,), jnp.int32)
        for g in range(4):
            def p1(j, cr, g=g):
                s0, s1, m0, m1, base = cr
                vs, ms, css, pcs = [], [], [], []
                for u in range(_U):
                    v = rowbuf[pl.ds(par + (g * 128 + j * _U + u) * 16, 16)]
                    vs.append(v)
                    m = v >= bprev
                    ms.append(m)
                    css.append(plsc.cumsum(m.astype(jnp.int32)))
                    pcs.append(plsc.all_reduce_population_count(m))
                    if u % 2 == 0:
                        s0 = s0 + v * v
                        m0 = jnp.maximum(m0, v)
                    else:
                        s1 = s1 + v * v
                        m1 = jnp.maximum(m1, v)
                for u in range(_U):
                    plsc.store_scatter(keybuf, [base + css[u] - 1],
                                       _tokey(vs[u]), mask=ms[u])
                    base = base + pcs[u]
                return s0, s1, m0, m1, base
            gm0 = jnp.full((16,), -jnp.inf, jnp.float32)
            s0, s1, m0, m1, off = lax.fori_loop(
                0, 128 // _U, p1, (ssqs[0], ssqs[1], gm0, gm0, off))
            ssqs = [s0, s1]
            gms.append(jnp.maximum(m0, m1))
        gmin = jnp.minimum(jnp.minimum(gms[0], gms[1]),
                           jnp.minimum(gms[2], gms[3]))
        gmax = jnp.maximum(jnp.maximum(gms[0], gms[1]),
                           jnp.maximum(gms[2], gms[3]))
        lval = -jnp.max(-gmin)
        mval = jnp.max(gmax)
        s2_row = jnp.sum(ssqs[0] + ssqs[1])
        nc_spec = jnp.max(off)

        def p2fb(j, off2):
            vs, ms, css, pcs = [], [], [], []
            for u in range(_U):
                v = rowbuf[pl.ds(par + (j * _U + u) * 16, 16)]
                m = v >= lval
                vs.append(v)
                ms.append(m)
                css.append(plsc.cumsum(m.astype(jnp.int32)))
                pcs.append(plsc.all_reduce_population_count(m))
            base = off2
            for u in range(_U):
                plsc.store_scatter(keybuf, [base + css[u] - 1],
                                   _tokey(vs[u]), mask=ms[u])
                base = base + pcs[u]
            return base

        def fb(_):
            off2 = lax.fori_loop(0, _NVEC // _U, p2fb,
                                 jnp.zeros((16,), jnp.int32))
            return jnp.max(off2), lval

        nc, lo_val = lax.cond(nc_spec < _TOPK, fb,
                              lambda _: (nc_spec, bprev), None)

        sent = jnp.full((16,), jnp.int32(_INT_MIN))
        lane16 = lax.broadcasted_iota(jnp.int32, (16,), 0)
        tmask = jnp.ones((16,), jnp.bool_)
        for u in range(4):
            plsc.store_scatter(keybuf, [nc + u * 16 + lane16], sent,
                               mask=tmask)
        nv4 = (nc + 63) >> 6

        klo = _tokey(lo_val)
        khi = _tokey(mval)
        u_lo = klo ^ jnp.int32(_INT_MIN)
        u_hi = khi ^ jnp.int32(_INT_MIN)
        sm = u_lo ^ u_hi
        sm = sm | lax.shift_right_logical(sm, 1)
        sm = sm | lax.shift_right_logical(sm, 2)
        sm = sm | lax.shift_right_logical(sm, 4)
        sm = sm | lax.shift_right_logical(sm, 8)
        sm = sm | lax.shift_right_logical(sm, 16)
        sp1 = sm + 1
        expo = lax.shift_right_logical(
            lax.bitcast_convert_type(sp1.astype(jnp.float32), jnp.int32),
            23) - 127
        nbits = jnp.where(sm < 0, jnp.int32(32),
                          jnp.where(sm == 0, jnp.int32(0), expo))

        def sbody(i, u):
            bit = jnp.full((16,), jnp.int32(1)) << (nbits - 1 - i)
            cand = u | bit
            thr = cand ^ jnp.int32(_INT_MIN)

            def cbody(jj, cr):
                a0, a1 = cr
                k0 = keybuf[pl.ds((jj * 4) * 16, 16)]
                k1 = keybuf[pl.ds((jj * 4 + 1) * 16, 16)]
                k2 = keybuf[pl.ds((jj * 4 + 2) * 16, 16)]
                k3 = keybuf[pl.ds((jj * 4 + 3) * 16, 16)]
                a0 = a0 + plsc.all_reduce_population_count(k0 >= thr) \
                    + plsc.all_reduce_population_count(k2 >= thr)
                a1 = a1 + plsc.all_reduce_population_count(k1 >= thr) \
                    + plsc.all_reduce_population_count(k3 >= thr)
                return a0, a1
            z = jnp.zeros((16,), jnp.int32)
            a0, a1 = lax.fori_loop(0, nv4, cbody, (z, z))
            cnt = a0 + a1
            return jnp.where(cnt >= _TOPK, cand, u)

        u_splat = lax.fori_loop(0, nbits, sbody,
                                jnp.full((16,), u_hi & ~sm))
        t_key = jnp.max(u_splat) ^ jnp.int32(_INT_MIN)

        def fbody(jj, cr):
            cgt, sgt = cr
            for u4 in range(4):
                kv = keybuf[pl.ds((jj * 4 + u4) * 16, 16)]
                gt = kv > t_key
                fv = _toval(kv)
                sgt = sgt + jnp.where(gt, fv * fv, 0.0)
                cgt = cgt + gt.astype(jnp.int32)
            return cgt, sgt
        cgt, sgt = lax.fori_loop(
            0, nv4, fbody,
            (jnp.zeros((16,), jnp.int32), jnp.zeros((16,), jnp.float32)))
        cnt_gt = jnp.sum(cgt)
        ssq_gt = jnp.sum(sgt)
        tval = _toval(t_key)
        top_row = ssq_gt + (_TOPK - cnt_gt).astype(jnp.float32) * tval * tval
        return s2_acc + s2_row, top_acc + top_row, lval

    s2_acc, top_acc, _ = lax.fori_loop(
        0, _RPW, row_body,
        (jnp.float32(0.0), jnp.float32(0.0), jnp.float32(jnp.inf)))

    lane = lax.broadcasted_iota(jnp.int32, (16,), 0)
    ov = jnp.where(lane == 0, s2_acc, jnp.where(lane == 1, top_acc, 0.0))
    outvec[...] = ov
    pltpu.sync_copy(outvec, out_hbm.at[wid])


_sc_topk = functools.partial(
    pl.kernel,
    mesh=plsc.VectorSubcoreMesh(core_axis_name="c", subcore_axis_name="s"),
    out_type=jax.ShapeDtypeStruct((_NW, 16), jnp.float32),
    scratch_types=[
        pltpu.VMEM((2 * _COLS,), jnp.float32),
        pltpu.VMEM((_COLS + 64,), jnp.int32),
        pltpu.VMEM((16,), jnp.float32),
        pltpu.SemaphoreType.DMA,
    ],
    compiler_params=pltpu.CompilerParams(needs_layout_passes=False),
)(_sc_body)


def _count16(mask16):
    m = mask16
    while m.shape[1] > 128:
        h = m.shape[1] // 2
        m = m[:, :h] + m[:, h:]
    return jnp.sum(m.astype(jnp.int32), axis=1, keepdims=True)


def _tc_block(x_ref, y_ref, a_ref, rec_ref, s2_ref, top_ref):
    i = pl.program_id(0)
    d = x_ref[...] - y_ref[...]
    rec_ref[...] = jnp.where(i < 4, jnp.sum(d * d), 0.0).reshape(1, 1, 1)

    a = a_ref[...]
    sq = a * a
    s2_ref[...] = jnp.sum(sq).reshape(1, 1, 1)

    b = jax.lax.bitcast_convert_type(a, jnp.int32)
    key = jnp.where(b >= 0, b, b ^ _MASK)

    key3 = key.reshape(a.shape[0], _COLS // 128, 128)
    cmax = jnp.max(key3, axis=1)
    lo_key = jnp.min(cmax)
    hi_key = jnp.max(cmax)
    u_lo = lo_key ^ jnp.int32(_INT_MIN)
    u_hi = hi_key ^ jnp.int32(_INT_MIN)
    sm = u_lo ^ u_hi
    sm = sm | jax.lax.shift_right_logical(sm, 1)
    sm = sm | jax.lax.shift_right_logical(sm, 2)
    sm = sm | jax.lax.shift_right_logical(sm, 4)
    sm = sm | jax.lax.shift_right_logical(sm, 8)
    sm = sm | jax.lax.shift_right_logical(sm, 16)
    sp1 = sm + 1
    expo = jax.lax.shift_right_logical(
        jax.lax.bitcast_convert_type(sp1.astype(jnp.float32), jnp.int32),
        23) - 127
    nbits = jnp.where(sm < 0, jnp.int32(32),
                      jnp.where(sm == 0, jnp.int32(0), expo))
    u_start = u_hi & ~sm

    k16 = (key >> 16).astype(jnp.int16)
    nh = jnp.maximum(nbits - 16, 0)
    uh_start = jax.lax.shift_right_logical(u_start, 16)

    def body_hi(i, uh):
        bit = jnp.int32(1) << (nh - 1 - i)
        cand = uh | bit
        t16c = (cand ^ 0x8000).astype(jnp.int16)
        cnt = _count16((k16 >= t16c).astype(jnp.int16))
        return jnp.where(cnt >= _TOPK, cand, uh)

    uh0 = jnp.full((a.shape[0], 1), uh_start, jnp.int32)
    uh = jax.lax.fori_loop(0, nh, body_hi, uh0)
    t16 = (uh ^ 0x8000).astype(jnp.int16)

    c_gt_hi = _count16((k16 > t16).astype(jnp.int16))
    r = _TOPK - c_gt_hi

    l16s = key.astype(jnp.int16) ^ jnp.int16(-(2 ** 15))
    m16 = jnp.where(k16 == t16, l16s, jnp.int16(-(2 ** 15)))
    nl = jnp.minimum(nbits, 16)
    ul_start = u_start & 0xFFFF & ~((jnp.int32(1) << nl) - 1)

    def body_lo(i, ul):
        bit = jnp.int32(1) << (nl - 1 - i)
        cand = ul | bit
        ts = (cand ^ 0x8000).astype(jnp.int16)
        cnt = _count16((m16 >= ts).astype(jnp.int16))
        return jnp.where(cnt >= r, cand, ul)

    ul0 = jnp.full((a.shape[0], 1), ul_start, jnp.int32)
    ul = jax.lax.fori_loop(0, nl, body_lo, ul0)

    t_key = (jax.lax.convert_element_type(t16, jnp.int32) << 16) | ul

    gt = key > t_key
    cnt_gt = jnp.sum(gt.astype(jnp.float32), axis=1, keepdims=True)
    ssq_gt = jnp.sum(jnp.where(gt, sq, 0.0), axis=1, keepdims=True)
    t_val = _toval(t_key)
    top = ssq_gt + (_TOPK - cnt_gt) * t_val * t_val
    top_ref[...] = jnp.sum(top).reshape(1, 1, 1)


def kernel(x, y, attn):
    sc_parts = _sc_topk(attn)
    rec_p, s2_p, top_p = pl.pallas_call(
        _tc_block,
        grid=(_GRID,),
        in_specs=[
            pl.BlockSpec((_ROWS // 4, _XC),
                         lambda i: (jnp.minimum(i, 3), 0)),
            pl.BlockSpec((_ROWS // 4, _XC),
                         lambda i: (jnp.minimum(i, 3), 0)),
            pl.BlockSpec((_BLOCK_R, _COLS),
                         lambda i: (i + _SC_ROWS // _BLOCK_R, 0)),
        ],
        out_specs=[
            pl.BlockSpec((1, 1, 1), lambda i: (i, 0, 0)),
            pl.BlockSpec((1, 1, 1), lambda i: (i, 0, 0)),
            pl.BlockSpec((1, 1, 1), lambda i: (i, 0, 0)),
        ],
        out_shape=[
            jax.ShapeDtypeStruct((_GRID, 1, 1), jnp.float32),
            jax.ShapeDtypeStruct((_GRID, 1, 1), jnp.float32),
            jax.ShapeDtypeStruct((_GRID, 1, 1), jnp.float32),
        ],
    )(x, y, attn)
    rec = jnp.sum(rec_p) / (_ROWS * _XC)
    s2 = jnp.sum(s2_p) + jnp.sum(sc_parts[:, 0])
    top = jnp.sum(top_p) + jnp.sum(sc_parts[:, 1])
    attn_loss = (s2 - top) / (_ROWS * _COLS)
    return rec + 0.5 * attn_loss
